# baseline (speedup 1.0000x reference)
import jax
import jax.numpy as jnp
from jax.experimental import pallas as pl
from jax.experimental.pallas import tpu as pltpu

_T, _B, _NG, _NP = 100, 256, 2048, 512
_BB = 256
_NCHUNKS = 4


def _dot(a, b):
    return jax.lax.dot_general(
        a, b, (((1,), (0,)), ((), ())), preferred_element_type=jnp.float32
    )


def _enc_body(p0_ref, wenc_ref, h0_ref):
    h0_ref[...] = _dot(p0_ref[...], wenc_ref[...])


def _rnn_body(v_ref, h0_ref, wih_ref, whh_ref, wdec_ref, out_ref, h_ref):
    t = pl.program_id(1)

    @pl.when(t == 0)
    def _():
        h_ref[...] = h0_ref[...]

    h_prev = h_ref[...]
    out_ref[0] = _dot(h_prev, wdec_ref[...])

    vt = v_ref[0]
    nc = _NG // _NCHUNKS
    for i in range(_NCHUNKS):
        sl = slice(i * nc, (i + 1) * nc)
        vin = vt[:, 0:1] * wih_ref[0:1, sl] + vt[:, 1:2] * wih_ref[1:2, sl]
        h_ref[:, sl] = jnp.tanh(vin + _dot(h_prev, whh_ref[:, sl]))


def kernel(v, p0, W_enc, W_ih, W_hh, W_dec):
    wenc_t = W_enc.T
    wih_t = W_ih.T
    whh_t = W_hh.T
    wdec_t = W_dec.T

    h0 = pl.pallas_call(
        _enc_body,
        out_shape=jax.ShapeDtypeStruct((_B, _NG), jnp.float32),
        name="elman_rnn_encoder",
    )(p0, wenc_t)

    return pl.pallas_call(
        _rnn_body,
        out_shape=jax.ShapeDtypeStruct((_T, _B, _NP), jnp.float32),
        grid=(_B // _BB, _T + 1),
        in_specs=[
            pl.BlockSpec(
                (1, _BB, 2), lambda b, t: (jnp.minimum(t, _T - 1), b, 0)
            ),
            pl.BlockSpec((_BB, _NG), lambda b, t: (b, 0)),
            pl.BlockSpec((2, _NG), lambda b, t: (0, 0)),
            pl.BlockSpec((_NG, _NG), lambda b, t: (0, 0)),
            pl.BlockSpec((_NG, _NP), lambda b, t: (0, 0)),
        ],
        out_specs=pl.BlockSpec(
            (1, _BB, _NP), lambda b, t: (jnp.maximum(t - 1, 0), b, 0)
        ),
        scratch_shapes=[pltpu.VMEM((_BB, _NG), jnp.float32)],
        compiler_params=pltpu.CompilerParams(
            dimension_semantics=("parallel", "arbitrary"),
            vmem_limit_bytes=56 * 1024 * 1024,
        ),
        name="elman_rnn_fused",
    )(v, h0, wih_t, whh_t, wdec_t)


# speedup vs baseline: 1.0497x; 1.0497x over previous
import jax
import jax.numpy as jnp
from jax.experimental import pallas as pl
from jax.experimental.pallas import tpu as pltpu

_T, _B, _NG, _NP = 100, 256, 2048, 512
_BB = 256
_TI = _T // 2


def _dot(a, b):
    return jax.lax.dot_general(
        a, b, (((1,), (0,)), ((), ())), preferred_element_type=jnp.float32
    )


def _enc_body(p0_ref, wenc_ref, h0_ref):
    h0_ref[...] = _dot(p0_ref[...], wenc_ref[...])


def _step(v_row, wih_ref, whh_ref, h):
    vin = v_row[:, 0:1] * wih_ref[0:1, :] + v_row[:, 1:2] * wih_ref[1:2, :]
    return jnp.tanh(vin + _dot(h, whh_ref[...]))


def _rnn_body(v_ref, h0_ref, wih_ref, whh_ref, wdec_ref, out_ref, h_ref, ha_ref):
    i = pl.program_id(1)

    @pl.when(i == 0)
    def _():
        h_ref[...] = h0_ref[...]

    h_entry = h_ref[...]
    out_ref[0] = _dot(ha_ref[...], wdec_ref[...])
    out_ref[1] = _dot(h_entry, wdec_ref[...])

    ha = _step(v_ref[0], wih_ref, whh_ref, h_entry)
    ha_ref[...] = ha
    h_ref[...] = _step(v_ref[1], wih_ref, whh_ref, ha)


def kernel(v, p0, W_enc, W_ih, W_hh, W_dec):
    wenc_t = W_enc.T
    wih_t = W_ih.T
    whh_t = W_hh.T
    wdec_t = W_dec.T

    h0 = pl.pallas_call(
        _enc_body,
        out_shape=jax.ShapeDtypeStruct((_B, _NG), jnp.float32),
        name="elman_rnn_encoder",
    )(p0, wenc_t)

    return pl.pallas_call(
        _rnn_body,
        out_shape=jax.ShapeDtypeStruct((_T, _B, _NP), jnp.float32),
        grid=(_B // _BB, _TI + 1),
        in_specs=[
            pl.BlockSpec(
                (2, _BB, 2), lambda b, i: (jnp.minimum(i, _TI - 1), b, 0)
            ),
            pl.BlockSpec((_BB, _NG), lambda b, i: (b, 0)),
            pl.BlockSpec((2, _NG), lambda b, i: (0, 0)),
            pl.BlockSpec((_NG, _NG), lambda b, i: (0, 0)),
            pl.BlockSpec((_NG, _NP), lambda b, i: (0, 0)),
        ],
        out_specs=pl.BlockSpec(
            (2, _BB, _NP), lambda b, i: (jnp.maximum(i - 1, 0), b, 0)
        ),
        scratch_shapes=[
            pltpu.VMEM((_BB, _NG), jnp.float32),
            pltpu.VMEM((_BB, _NG), jnp.float32),
        ],
        compiler_params=pltpu.CompilerParams(
            dimension_semantics=("parallel", "arbitrary"),
            vmem_limit_bytes=56 * 1024 * 1024,
        ),
        name="elman_rnn_fused",
    )(v, h0, wih_t, whh_t, wdec_t)


# speedup vs baseline: 1.0590x; 1.0089x over previous
import jax
import jax.numpy as jnp
from jax.experimental import pallas as pl
from jax.experimental.pallas import tpu as pltpu

_T, _B, _NG, _NP = 100, 256, 2048, 512
_BB = 256
_S = 4
_TI = _T // _S


def _dot(a, b):
    return jax.lax.dot_general(
        a, b, (((1,), (0,)), ((), ())), preferred_element_type=jnp.float32
    )


def _enc_body(p0_ref, wenc_ref, h0_ref):
    h0_ref[...] = _dot(p0_ref[...], wenc_ref[...])


def _step(v_row, wih_ref, whh_ref, h):
    vin = v_row[:, 0:1] * wih_ref[0:1, :] + v_row[:, 1:2] * wih_ref[1:2, :]
    return jnp.tanh(vin + _dot(h, whh_ref[...]))


def _rnn_body(v_ref, h0_ref, wih_ref, whh_ref, wdec_ref, out_ref, hs_ref):
    i = pl.program_id(1)

    @pl.when(i == 0)
    def _():
        hs_ref[_S - 1] = h0_ref[...]

    for k in range(_S):
        out_ref[k] = _dot(hs_ref[k], wdec_ref[...])

    h = hs_ref[_S - 1]
    for k in range(_S):
        h = _step(v_ref[k], wih_ref, whh_ref, h)
        hs_ref[k] = h


def kernel(v, p0, W_enc, W_ih, W_hh, W_dec):
    wenc_t = W_enc.T
    wih_t = W_ih.T
    whh_t = W_hh.T
    wdec_t = W_dec.T

    h0 = pl.pallas_call(
        _enc_body,
        out_shape=jax.ShapeDtypeStruct((_B, _NG), jnp.float32),
        name="elman_rnn_encoder",
    )(p0, wenc_t)

    return pl.pallas_call(
        _rnn_body,
        out_shape=jax.ShapeDtypeStruct((_T, _B, _NP), jnp.float32),
        grid=(_B // _BB, _TI + 1),
        in_specs=[
            pl.BlockSpec(
                (_S, _BB, 2), lambda b, i: (jnp.minimum(i, _TI - 1), b, 0)
            ),
            pl.BlockSpec((_BB, _NG), lambda b, i: (b, 0)),
            pl.BlockSpec((2, _NG), lambda b, i: (0, 0)),
            pl.BlockSpec((_NG, _NG), lambda b, i: (0, 0)),
            pl.BlockSpec((_NG, _NP), lambda b, i: (0, 0)),
        ],
        out_specs=pl.BlockSpec(
            (_S, _BB, _NP), lambda b, i: (jnp.maximum(i - 1, 0), b, 0)
        ),
        scratch_shapes=[
            pltpu.VMEM((_S, _BB, _NG), jnp.float32),
        ],
        compiler_params=pltpu.CompilerParams(
            dimension_semantics=("parallel", "arbitrary"),
            vmem_limit_bytes=56 * 1024 * 1024,
        ),
        name="elman_rnn_fused",
    )(v, h0, wih_t, whh_t, wdec_t)


# speedup vs baseline: 1.0679x; 1.0084x over previous
import jax
import jax.numpy as jnp
from jax.experimental import pallas as pl
from jax.experimental.pallas import tpu as pltpu

_T, _B, _NG, _NP = 100, 256, 2048, 512
_BB = 256
_S = 4
_TI = _T // _S


def _dot(a, b):
    return jax.lax.dot_general(
        a, b, (((1,), (0,)), ((), ())), preferred_element_type=jnp.float32
    )


def _step(v_row, wih_ref, whh_ref, h):
    vin = v_row[:, 0:1] * wih_ref[0:1, :] + v_row[:, 1:2] * wih_ref[1:2, :]
    return jnp.tanh(vin + _dot(h, whh_ref[...]))


def _rnn_body(v_ref, p0_ref, wenc_ref, wih_ref, whh_ref, wdec_ref, out_ref, hs_ref):
    i = pl.program_id(1)

    @pl.when(i == 0)
    def _():
        hs_ref[_S - 1] = _dot(p0_ref[...], wenc_ref[...])

    for k in range(_S):
        out_ref[k] = _dot(hs_ref[k], wdec_ref[...])

    h = hs_ref[_S - 1]
    for k in range(_S):
        h = _step(v_ref[k], wih_ref, whh_ref, h)
        hs_ref[k] = h


def kernel(v, p0, W_enc, W_ih, W_hh, W_dec):
    wenc_t = W_enc.T
    wih_t = W_ih.T
    whh_t = W_hh.T
    wdec_t = W_dec.T

    return pl.pallas_call(
        _rnn_body,
        out_shape=jax.ShapeDtypeStruct((_T, _B, _NP), jnp.float32),
        grid=(_B // _BB, _TI + 1),
        in_specs=[
            pl.BlockSpec(
                (_S, _BB, 2), lambda b, i: (jnp.minimum(i, _TI - 1), b, 0)
            ),
            pl.BlockSpec((_BB, _NP), lambda b, i: (b, 0)),
            pl.BlockSpec((_NP, _NG), lambda b, i: (0, 0)),
            pl.BlockSpec((2, _NG), lambda b, i: (0, 0)),
            pl.BlockSpec((_NG, _NG), lambda b, i: (0, 0)),
            pl.BlockSpec((_NG, _NP), lambda b, i: (0, 0)),
        ],
        out_specs=pl.BlockSpec(
            (_S, _BB, _NP), lambda b, i: (jnp.maximum(i - 1, 0), b, 0)
        ),
        scratch_shapes=[
            pltpu.VMEM((_S, _BB, _NG), jnp.float32),
        ],
        compiler_params=pltpu.CompilerParams(
            dimension_semantics=("parallel", "arbitrary"),
            vmem_limit_bytes=56 * 1024 * 1024,
        ),
        name="elman_rnn_fused",
    )(v, p0, wenc_t, wih_t, whh_t, wdec_t)
